# baseline (device time: 49239 ns/iter reference)
import jax
import jax.numpy as jnp
from jax import lax
from jax.experimental import pallas as pl
from jax.experimental.pallas import tpu as pltpu

N_DEV = 4
N_HOPS = N_DEV - 1


def kernel(x, Wq, K_ext, V_ext, Wo):
    b_loc, sq, d_model = x.shape
    _, hpb_x_dh = Wq.shape
    _, skv, hq, dh = K_ext.shape
    hpb = hpb_x_dh // dh
    rows = b_loc * sq

    my = lax.axis_index("i")
    Kb = lax.dynamic_slice_in_dim(K_ext, my * b_loc, b_loc, axis=0)
    Vb = lax.dynamic_slice_in_dim(V_ext, my * b_loc, b_loc, axis=0)

    def body(x_ref, wq_ref, k_ref, v_ref, wo_ref, out_ref,
             wq_com, wo_com, ctx_scr,
             wq_send, wq_recv, wo_send, wo_recv):
        my_pos = lax.axis_index("i")
        left = lax.rem(my_pos + N_DEV - 1, N_DEV)
        right = lax.rem(my_pos + 1, N_DEV)

        barrier_sem = pltpu.get_barrier_semaphore()
        for nbr in (left, right):
            pl.semaphore_signal(
                barrier_sem, inc=1,
                device_id=(nbr,), device_id_type=pl.DeviceIdType.MESH,
            )
        pl.semaphore_wait(barrier_sem, 2)

        def make(src, dst, ssem, rsem):
            return pltpu.make_async_remote_copy(
                src_ref=src, dst_ref=dst, send_sem=ssem, recv_sem=rsem,
                device_id=(right,), device_id_type=pl.DeviceIdType.MESH,
            )

        r_wq = [make(wq_ref, wq_com.at[0], wq_send.at[0], wq_recv.at[0])]
        r_wo = [make(wo_ref, wo_com.at[0], wo_send.at[0], wo_recv.at[0])]
        r_wq[0].start()
        r_wo[0].start()

        x_flat = jnp.reshape(x_ref[...], (rows, d_model))

        ri = lax.broadcasted_iota(jnp.int32, (sq, skv), 0)
        ci = lax.broadcasted_iota(jnp.int32, (sq, skv), 1)
        qb = ri // 64
        kb = ci // 64
        mask = (qb == kb) | (kb == 0) | (((qb + kb) % 3) == 0)

        def compute_block(wq_blk, wo_blk, origin, acc):
            q = jnp.dot(x_flat, wq_blk, preferred_element_type=jnp.float32)
            for b in range(b_loc):
                q_b = q[b * sq:(b + 1) * sq, :]
                for hh in range(hpb):
                    hidx = origin * hpb + hh
                    qh = q_b[:, hh * dh:(hh + 1) * dh]
                    kh = k_ref[b, :, hidx, :]
                    vh = v_ref[b, :, hidx, :]
                    s = lax.dot_general(
                        qh, kh, (((1,), (1,)), ((), ())),
                        preferred_element_type=jnp.float32,
                    ) * 0.125
                    s = jnp.where(mask, s, -1e9)
                    m = jnp.max(s, axis=-1, keepdims=True)
                    w = jnp.exp(s - m)
                    w = w / jnp.sum(w, axis=-1, keepdims=True)
                    ch = jnp.dot(w, vh, preferred_element_type=jnp.float32)
                    ctx_scr[b * sq:(b + 1) * sq, hh * dh:(hh + 1) * dh] = ch
            contrib = jnp.dot(ctx_scr[...], wo_blk,
                              preferred_element_type=jnp.float32)
            return contrib if acc is None else acc + contrib

        acc = compute_block(wq_ref[...], wo_ref[...], my_pos, None)

        for h in range(N_HOPS):
            r_wq[h].wait_recv()
            r_wo[h].wait_recv()
            if h + 1 < N_HOPS:
                nq = make(wq_com.at[h], wq_com.at[h + 1],
                          wq_send.at[h + 1], wq_recv.at[h + 1])
                no = make(wo_com.at[h], wo_com.at[h + 1],
                          wo_send.at[h + 1], wo_recv.at[h + 1])
                nq.start()
                no.start()
                r_wq.append(nq)
                r_wo.append(no)
            origin = lax.rem(my_pos + N_DEV - 1 - h, N_DEV)
            acc = compute_block(wq_com[h], wo_com[h], origin, acc)

        for r in r_wq + r_wo:
            r.wait_send()

        for b in range(b_loc):
            out_ref[b, :, :] = acc[b * sq:(b + 1) * sq, :]

    return pl.pallas_call(
        body,
        out_shape=jax.ShapeDtypeStruct((b_loc, sq, d_model), jnp.float32),
        in_specs=[pl.BlockSpec(memory_space=pltpu.VMEM)] * 5,
        out_specs=pl.BlockSpec(memory_space=pltpu.VMEM),
        scratch_shapes=[
            pltpu.VMEM((N_HOPS, d_model, hpb * dh), jnp.float32),
            pltpu.VMEM((N_HOPS, hpb * dh, d_model), jnp.float32),
            pltpu.VMEM((rows, hpb * dh), jnp.float32),
            pltpu.SemaphoreType.DMA((N_HOPS,)),
            pltpu.SemaphoreType.DMA((N_HOPS,)),
            pltpu.SemaphoreType.DMA((N_HOPS,)),
            pltpu.SemaphoreType.DMA((N_HOPS,)),
        ],
        compiler_params=pltpu.CompilerParams(collective_id=0),
    )(x, Wq, Kb, Vb, Wo)


# device time: 34812 ns/iter; 1.4144x vs baseline; 1.4144x over previous
import jax
import jax.numpy as jnp
from jax import lax
from jax.experimental import pallas as pl
from jax.experimental.pallas import tpu as pltpu

N_DEV = 4
N_HOPS = N_DEV - 1


def kernel(x, Wq, K_ext, V_ext, Wo):
    b_loc, sq, d_model = x.shape
    _, hpb_x_dh = Wq.shape
    _, skv, hq, dh = K_ext.shape
    hpb = hpb_x_dh // dh
    rows = b_loc * sq

    my = lax.axis_index("i")
    Kb = lax.dynamic_slice_in_dim(K_ext, my * b_loc, b_loc, axis=0)
    Vb = lax.dynamic_slice_in_dim(V_ext, my * b_loc, b_loc, axis=0)

    def body(x_ref, wq_ref, k_ref, v_ref, wo_ref, out_ref,
             wq_com, wo_com, ctx_scr,
             wq_send, wq_recv, wo_send, wo_recv):
        my_pos = lax.axis_index("i")

        barrier_sem = pltpu.get_barrier_semaphore()
        for d in (1, 2, 3):
            pl.semaphore_signal(
                barrier_sem, inc=1,
                device_id=(lax.rem(my_pos + d, N_DEV),),
                device_id_type=pl.DeviceIdType.MESH,
            )
        pl.semaphore_wait(barrier_sem, N_DEV - 1)

        def make(src, dst, ssem, rsem, dev):
            return pltpu.make_async_remote_copy(
                src_ref=src, dst_ref=dst, send_sem=ssem, recv_sem=rsem,
                device_id=(dev,), device_id_type=pl.DeviceIdType.MESH,
            )

        sends = []
        for d in (1, 3, 2):
            s = 3 - d
            dev = lax.rem(my_pos + d, N_DEV)
            rq = make(wq_ref, wq_com.at[s], wq_send.at[d - 1],
                      wq_recv.at[s], dev)
            ro = make(wo_ref, wo_com.at[s], wo_send.at[d - 1],
                      wo_recv.at[s], dev)
            rq.start()
            ro.start()
            sends += [rq, ro]

        x_flat = jnp.reshape(x_ref[...], (rows, d_model))

        ri = lax.broadcasted_iota(jnp.int32, (sq, skv), 0)
        ci = lax.broadcasted_iota(jnp.int32, (sq, skv), 1)
        qb = ri // 64
        kb = ci // 64
        mask = (qb == kb) | (kb == 0) | (((qb + kb) % 3) == 0)

        def compute_block(wq_blk, wo_blk, origin, acc):
            q = jnp.dot(x_flat, wq_blk, preferred_element_type=jnp.float32)
            for b in range(b_loc):
                q_b = q[b * sq:(b + 1) * sq, :]
                for hh in range(hpb):
                    hidx = origin * hpb + hh
                    qh = q_b[:, hh * dh:(hh + 1) * dh]
                    kh = k_ref[b, :, hidx, :]
                    vh = v_ref[b, :, hidx, :]
                    s = lax.dot_general(
                        qh, kh, (((1,), (1,)), ((), ())),
                        preferred_element_type=jnp.float32,
                    ) * 0.125
                    s = jnp.where(mask, s, -1e9)
                    m = jnp.max(s, axis=-1, keepdims=True)
                    w = jnp.exp(s - m)
                    w = w / jnp.sum(w, axis=-1, keepdims=True)
                    ch = jnp.dot(w, vh, preferred_element_type=jnp.float32)
                    ctx_scr[b * sq:(b + 1) * sq, hh * dh:(hh + 1) * dh] = ch
            contrib = jnp.dot(ctx_scr[...], wo_blk,
                              preferred_element_type=jnp.float32)
            return contrib if acc is None else acc + contrib

        acc = compute_block(wq_ref[...], wo_ref[...], my_pos, None)

        for s in (0, 2, 1):
            rq = make(wq_ref, wq_com.at[s], wq_send.at[s],
                      wq_recv.at[s], my_pos)
            ro = make(wo_ref, wo_com.at[s], wo_send.at[s],
                      wo_recv.at[s], my_pos)
            rq.wait_recv()
            ro.wait_recv()
            origin = lax.rem(my_pos + s + 1, N_DEV)
            acc = compute_block(wq_com[s], wo_com[s], origin, acc)

        for r in sends:
            r.wait_send()

        for b in range(b_loc):
            out_ref[b, :, :] = acc[b * sq:(b + 1) * sq, :]

    return pl.pallas_call(
        body,
        out_shape=jax.ShapeDtypeStruct((b_loc, sq, d_model), jnp.float32),
        in_specs=[pl.BlockSpec(memory_space=pltpu.VMEM)] * 5,
        out_specs=pl.BlockSpec(memory_space=pltpu.VMEM),
        scratch_shapes=[
            pltpu.VMEM((N_HOPS, d_model, hpb * dh), jnp.float32),
            pltpu.VMEM((N_HOPS, hpb * dh, d_model), jnp.float32),
            pltpu.VMEM((rows, hpb * dh), jnp.float32),
            pltpu.SemaphoreType.DMA((N_HOPS,)),
            pltpu.SemaphoreType.DMA((N_HOPS,)),
            pltpu.SemaphoreType.DMA((N_HOPS,)),
            pltpu.SemaphoreType.DMA((N_HOPS,)),
        ],
        compiler_params=pltpu.CompilerParams(collective_id=0),
    )(x, Wq, Kb, Vb, Wo)


# device time: 23928 ns/iter; 2.0578x vs baseline; 1.4549x over previous
import jax
import jax.numpy as jnp
from jax import lax
from jax.experimental import pallas as pl
from jax.experimental.pallas import tpu as pltpu

N_DEV = 4
N_HOPS = N_DEV - 1


def kernel(x, Wq, K_ext, V_ext, Wo):
    b_loc, sq, d_model = x.shape
    _, hpb_x_dh = Wq.shape
    _, skv, hq, dh = K_ext.shape
    hpb = hpb_x_dh // dh
    rows = b_loc * sq

    my = lax.axis_index("i")
    Kb = lax.dynamic_slice_in_dim(K_ext, my * b_loc, b_loc, axis=0)
    Vb = lax.dynamic_slice_in_dim(V_ext, my * b_loc, b_loc, axis=0)
    Wq = Wq.astype(jnp.bfloat16)
    Wo = Wo.astype(jnp.bfloat16)

    def body(x_ref, wq_ref, k_ref, v_ref, wo_ref, out_ref,
             wq_com, wo_com, ctx_scr,
             wq_send, wq_recv, wo_send, wo_recv):
        my_pos = lax.axis_index("i")

        barrier_sem = pltpu.get_barrier_semaphore()
        for d in (1, 2, 3):
            pl.semaphore_signal(
                barrier_sem, inc=1,
                device_id=(lax.rem(my_pos + d, N_DEV),),
                device_id_type=pl.DeviceIdType.MESH,
            )
        pl.semaphore_wait(barrier_sem, N_DEV - 1)

        def make(src, dst, ssem, rsem, dev):
            return pltpu.make_async_remote_copy(
                src_ref=src, dst_ref=dst, send_sem=ssem, recv_sem=rsem,
                device_id=(dev,), device_id_type=pl.DeviceIdType.MESH,
            )

        sends = []
        for d in (1, 3, 2):
            s = 3 - d
            dev = lax.rem(my_pos + d, N_DEV)
            rq = make(wq_ref, wq_com.at[s], wq_send.at[d - 1],
                      wq_recv.at[s], dev)
            ro = make(wo_ref, wo_com.at[s], wo_send.at[d - 1],
                      wo_recv.at[s], dev)
            rq.start()
            ro.start()
            sends += [rq, ro]

        x_flat = jnp.reshape(x_ref[...], (rows, d_model)).astype(jnp.bfloat16)

        ri = lax.broadcasted_iota(jnp.int32, (sq, skv), 0)
        ci = lax.broadcasted_iota(jnp.int32, (sq, skv), 1)
        qb = ri // 64
        kb = ci // 64
        mask = (qb == kb) | (kb == 0) | (((qb + kb) % 3) == 0)

        def compute_block(wq_blk, wo_blk, origin, acc):
            q = jnp.dot(x_flat, wq_blk, preferred_element_type=jnp.float32)
            q = q.astype(jnp.bfloat16)
            for b in range(b_loc):
                q_b = q[b * sq:(b + 1) * sq, :]
                for hh in range(hpb):
                    hidx = origin * hpb + hh
                    qh = q_b[:, hh * dh:(hh + 1) * dh]
                    kh = k_ref[b, :, hidx, :].astype(jnp.bfloat16)
                    vh = v_ref[b, :, hidx, :].astype(jnp.bfloat16)
                    s = lax.dot_general(
                        qh, kh, (((1,), (1,)), ((), ())),
                        preferred_element_type=jnp.float32,
                    ) * 0.125
                    s = jnp.where(mask, s, -1e9)
                    m = jnp.max(s, axis=-1, keepdims=True)
                    w = jnp.exp(s - m)
                    w = (w / jnp.sum(w, axis=-1, keepdims=True)
                         ).astype(jnp.bfloat16)
                    ch = jnp.dot(w, vh, preferred_element_type=jnp.float32)
                    ctx_scr[b * sq:(b + 1) * sq, hh * dh:(hh + 1) * dh] = (
                        ch.astype(jnp.bfloat16))
            contrib = jnp.dot(ctx_scr[...], wo_blk,
                              preferred_element_type=jnp.float32)
            return contrib if acc is None else acc + contrib

        acc = compute_block(wq_ref[...], wo_ref[...], my_pos, None)

        for s in (0, 2, 1):
            rq = make(wq_ref, wq_com.at[s], wq_send.at[s],
                      wq_recv.at[s], my_pos)
            ro = make(wo_ref, wo_com.at[s], wo_send.at[s],
                      wo_recv.at[s], my_pos)
            rq.wait_recv()
            ro.wait_recv()
            origin = lax.rem(my_pos + s + 1, N_DEV)
            acc = compute_block(wq_com[s], wo_com[s], origin, acc)

        for r in sends:
            r.wait_send()

        for b in range(b_loc):
            out_ref[b, :, :] = acc[b * sq:(b + 1) * sq, :]

    return pl.pallas_call(
        body,
        out_shape=jax.ShapeDtypeStruct((b_loc, sq, d_model), jnp.float32),
        in_specs=[pl.BlockSpec(memory_space=pltpu.VMEM)] * 5,
        out_specs=pl.BlockSpec(memory_space=pltpu.VMEM),
        scratch_shapes=[
            pltpu.VMEM((N_HOPS, d_model, hpb * dh), jnp.bfloat16),
            pltpu.VMEM((N_HOPS, hpb * dh, d_model), jnp.bfloat16),
            pltpu.VMEM((rows, hpb * dh), jnp.bfloat16),
            pltpu.SemaphoreType.DMA((N_HOPS,)),
            pltpu.SemaphoreType.DMA((N_HOPS,)),
            pltpu.SemaphoreType.DMA((N_HOPS,)),
            pltpu.SemaphoreType.DMA((N_HOPS,)),
        ],
        compiler_params=pltpu.CompilerParams(collective_id=0),
    )(x, Wq, Kb, Vb, Wo)
